# baseline (speedup 1.0000x reference)
import functools

import jax
import jax.numpy as jnp
from jax import lax
from jax.experimental import pallas as pl
from jax.experimental.pallas import tpu as pltpu
from jax.experimental.pallas import tpu_sc as plsc

VOCAB = 1_000_000
EMBED = 300
SEQ = 60
BATCH = 16384

NC = 2
NS = 16
L = 16
NW = NC * NS
SENT_PER_W = BATCH // NW
GROUPS_PER_W = SENT_PER_W // L
IDX_PER_W = SENT_PER_W * SEQ
CHUNK = 128
NCHUNK = IDX_PER_W // CHUNK
FLIGHT = 8


def _fold_weights(W_in, b_in, W_fc, b_fc):
    def body(wi_ref, bi_ref, wf_ref, bf_ref, v_ref, c_ref):
        wf = wf_ref[...]
        v = jnp.dot(wi_ref[...], wf,
                    preferred_element_type=jnp.float32)
        v_ref[...] = v * (1.0 / SEQ)
        c = jnp.sum(bi_ref[...] * wf[:, 0]) + bf_ref[0]
        c_ref[...] = jnp.broadcast_to(c, (L,))

    return pl.pallas_call(
        body,
        out_shape=(jax.ShapeDtypeStruct((EMBED, 1), jnp.float32),
                   jax.ShapeDtypeStruct((L,), jnp.float32)),
    )(W_in, b_in, W_fc, b_fc)


def _matvec(tableT, v_col):
    C = 8192
    G = -(-VOCAB // C)

    def body(t_ref, v_ref, s_ref):
        s_ref[...] = jnp.sum(t_ref[...] * v_ref[...], axis=0)

    return pl.pallas_call(
        body,
        grid=(G,),
        in_specs=[pl.BlockSpec((EMBED, C), lambda i: (0, i)),
                  pl.BlockSpec((EMBED, 1), lambda i: (0, 0))],
        out_specs=pl.BlockSpec((C,), lambda i: (i,)),
        out_shape=jax.ShapeDtypeStruct((VOCAB,), jnp.float32),
        compiler_params=pltpu.CompilerParams(
            dimension_semantics=("parallel",)),
    )(tableT, v_col)


def _gather_reduce(scores, xt, c16):
    mesh = plsc.VectorSubcoreMesh(core_axis_name="c", subcore_axis_name="s")

    @functools.partial(
        pl.kernel,
        out_type=jax.ShapeDtypeStruct((BATCH,), jnp.float32),
        mesh=mesh,
        scratch_types=[
            pltpu.VMEM((NCHUNK, CHUNK), jnp.int32),
            pltpu.VMEM((IDX_PER_W,), jnp.float32),
            pltpu.VMEM((SENT_PER_W,), jnp.float32),
            pltpu.VMEM((L,), jnp.float32),
            pltpu.SemaphoreType.DMA,
        ],
    )
    def k(scores_hbm, xt_hbm, c_hbm, out_hbm, idx_v, g_v, out_v, c_v, sem):
        wid = lax.axis_index("s") * NC + lax.axis_index("c")
        pltpu.sync_copy(c_hbm, c_v)
        pltpu.sync_copy(xt_hbm.at[wid], idx_v)

        @pl.loop(0, NCHUNK, step=FLIGHT)
        def _(o):
            copies = [
                pltpu.async_copy(
                    scores_hbm.at[idx_v.at[o + b]],
                    g_v.at[pl.ds((o + b) * CHUNK, CHUNK)],
                    sem,
                )
                for b in range(FLIGHT)
            ]
            for cp in copies:
                cp.wait()

        cvec = c_v[...]

        @pl.loop(0, GROUPS_PER_W)
        def _(gr):
            def body(w, acc):
                return acc + g_v[pl.ds(gr * (SEQ * L) + w * L, L)]
            acc = lax.fori_loop(0, SEQ, body, jnp.zeros((L,), jnp.float32))
            t = acc + cvec
            out_v[pl.ds(gr * L, L)] = 1.0 / (1.0 + jnp.exp(-t))

        pltpu.sync_copy(out_v, out_hbm.at[pl.ds(wid * SENT_PER_W, SENT_PER_W)])

    return k(scores, xt, c16)


def kernel(x, table, W_in, b_in, W_fc, b_fc):
    v_col, c16 = _fold_weights(W_in, b_in, W_fc, b_fc)
    scores = _matvec(table.T, v_col)
    xt = (x.reshape(BATCH // L, L, SEQ)
            .transpose(0, 2, 1)
            .reshape(NW, NCHUNK, CHUNK))
    return _gather_reduce(scores, xt, c16)


# speedup vs baseline: 1.1398x; 1.1398x over previous
import functools

import jax
import jax.numpy as jnp
from jax import lax
from jax.experimental import pallas as pl
from jax.experimental.pallas import tpu as pltpu
from jax.experimental.pallas import tpu_sc as plsc

VOCAB = 1_000_000
EMBED = 300
SEQ = 60
BATCH = 16384

NC = 2
NS = 16
L = 16
NW = NC * NS
SENT_PER_W = BATCH // NW
GROUPS_PER_W = SENT_PER_W // L
IDX_PER_W = SENT_PER_W * SEQ
CHUNK = 128
CPW = SENT_PER_W // CHUNK
NCHUNK = IDX_PER_W // CHUNK
FLIGHT = 16


def _fold_weights(W_in, b_in, W_fc, b_fc):
    def body(wi_ref, bi_ref, wf_ref, bf_ref, v_ref, c_ref):
        wf = wf_ref[...]
        v = jnp.dot(wi_ref[...], wf,
                    preferred_element_type=jnp.float32)
        v_ref[...] = v * (1.0 / SEQ)
        c = jnp.sum(bi_ref[...] * wf[:, 0]) + bf_ref[0]
        c_ref[...] = jnp.broadcast_to(c, (L,))

    return pl.pallas_call(
        body,
        out_shape=(jax.ShapeDtypeStruct((EMBED, 1), jnp.float32),
                   jax.ShapeDtypeStruct((L,), jnp.float32)),
    )(W_in, b_in, W_fc, b_fc)


def _matvec(tableT, v_col):
    C = 16384
    G = -(-VOCAB // C)

    def body(t_ref, v_ref, s_ref):
        s_ref[...] = jnp.sum(t_ref[...] * v_ref[...], axis=0)

    return pl.pallas_call(
        body,
        grid=(G,),
        in_specs=[pl.BlockSpec((EMBED, C), lambda i: (0, i)),
                  pl.BlockSpec((EMBED, 1), lambda i: (0, 0))],
        out_specs=pl.BlockSpec((C,), lambda i: (i,)),
        out_shape=jax.ShapeDtypeStruct((VOCAB,), jnp.float32),
        compiler_params=pltpu.CompilerParams(
            dimension_semantics=("parallel",)),
    )(tableT, v_col)


def _gather_reduce(scores, xT, c16):
    mesh = plsc.VectorSubcoreMesh(core_axis_name="c", subcore_axis_name="s")

    @functools.partial(
        pl.kernel,
        out_type=jax.ShapeDtypeStruct((BATCH,), jnp.float32),
        mesh=mesh,
        scratch_types=[
            pltpu.VMEM((SEQ, SENT_PER_W), jnp.int32),
            pltpu.VMEM((IDX_PER_W,), jnp.float32),
            pltpu.VMEM((SENT_PER_W,), jnp.float32),
            pltpu.VMEM((L,), jnp.float32),
            pltpu.SemaphoreType.DMA,
        ],
    )
    def k(scores_hbm, xT_hbm, c_hbm, out_hbm, idx_v, g_v, out_v, c_v, sem):
        wid = lax.axis_index("s") * NC + lax.axis_index("c")
        pltpu.sync_copy(c_hbm, c_v)
        pltpu.sync_copy(xT_hbm.at[:, pl.ds(wid * SENT_PER_W, SENT_PER_W)],
                        idx_v)

        @pl.loop(0, NCHUNK, step=FLIGHT)
        def _(o):
            copies = []
            for b in range(FLIGHT):
                t = o + b
                w, c = t // CPW, t % CPW
                copies.append(pltpu.async_copy(
                    scores_hbm.at[idx_v.at[w, pl.ds(c * CHUNK, CHUNK)]],
                    g_v.at[pl.ds(t * CHUNK, CHUNK)],
                    sem,
                ))
            for cp in copies:
                cp.wait()

        cvec = c_v[...]

        @pl.loop(0, GROUPS_PER_W)
        def _(gr):
            acc = cvec
            for w in range(SEQ):
                acc = acc + g_v[pl.ds(gr * L + w * SENT_PER_W, L)]
            out_v[pl.ds(gr * L, L)] = 1.0 / (1.0 + jnp.exp(-acc))

        pltpu.sync_copy(out_v, out_hbm.at[pl.ds(wid * SENT_PER_W, SENT_PER_W)])

    return k(scores, xT, c16)


def kernel(x, table, W_in, b_in, W_fc, b_fc):
    v_col, c16 = _fold_weights(W_in, b_in, W_fc, b_fc)
    scores = _matvec(table.T, v_col)
    return _gather_reduce(scores, x.T, c16)


# speedup vs baseline: 1.2156x; 1.0665x over previous
import functools

import jax
import jax.numpy as jnp
from jax import lax
from jax.experimental import pallas as pl
from jax.experimental.pallas import tpu as pltpu
from jax.experimental.pallas import tpu_sc as plsc

VOCAB = 1_000_000
EMBED = 300
SEQ = 60
BATCH = 16384

NC = 2
NS = 16
L = 16
NW = NC * NS
SENT_PER_W = BATCH // NW
GROUPS_PER_W = SENT_PER_W // L
IDX_PER_W = SENT_PER_W * SEQ
CHUNK = 128
CPW = SENT_PER_W // CHUNK
NCHUNK = IDX_PER_W // CHUNK
FLIGHT = 16


def _fold_weights(W_in, b_in, W_fc, b_fc):
    def body(wi_ref, bi_ref, wf_ref, bf_ref, v_ref, c_ref):
        wf = wf_ref[...]
        v = jnp.dot(wi_ref[...], wf,
                    preferred_element_type=jnp.float32)
        v_ref[...] = v * (1.0 / SEQ)
        c = jnp.sum(bi_ref[...] * wf[:, 0]) + bf_ref[0]
        c_ref[...] = jnp.broadcast_to(c, (L,))

    return pl.pallas_call(
        body,
        out_shape=(jax.ShapeDtypeStruct((EMBED, 1), jnp.float32),
                   jax.ShapeDtypeStruct((L,), jnp.float32)),
    )(W_in, b_in, W_fc, b_fc)


def _matvec(tableT, v_col):
    C = 16384
    G = -(-VOCAB // C)

    def body(t_ref, v_ref, s_ref):
        s_ref[...] = jnp.sum(t_ref[...] * v_ref[...], axis=0)

    return pl.pallas_call(
        body,
        grid=(G,),
        in_specs=[pl.BlockSpec((EMBED, C), lambda i: (0, i)),
                  pl.BlockSpec((EMBED, 1), lambda i: (0, 0))],
        out_specs=pl.BlockSpec((C,), lambda i: (i,)),
        out_shape=jax.ShapeDtypeStruct((VOCAB,), jnp.float32),
        compiler_params=pltpu.CompilerParams(
            dimension_semantics=("parallel",)),
    )(tableT, v_col)


def _gather_reduce(scores, xT, c16):
    mesh = plsc.VectorSubcoreMesh(core_axis_name="c", subcore_axis_name="s")

    @functools.partial(
        pl.kernel,
        out_type=jax.ShapeDtypeStruct((BATCH,), jnp.float32),
        mesh=mesh,
        scratch_types=[
            pltpu.VMEM((SEQ, SENT_PER_W), jnp.int32),
            pltpu.VMEM((IDX_PER_W,), jnp.float32),
            pltpu.VMEM((SENT_PER_W,), jnp.float32),
            pltpu.VMEM((L,), jnp.float32),
            pltpu.VMEM_SHARED((VOCAB,), jnp.float32),
            pltpu.SemaphoreType.DMA,
        ],
    )
    def k(scores_hbm, xT_hbm, c_hbm, out_hbm, idx_v, g_v, out_v, c_v,
          s_sh, sem):
        sid = lax.axis_index("s")
        wid = sid * NC + lax.axis_index("c")
        pltpu.sync_copy(c_hbm, c_v)

        CH = 15624
        for kk in range(4):
            cix = sid + NS * kk

            @pl.when(cix < 63)
            def _():
                off = cix * CH
                pltpu.sync_copy(scores_hbm.at[pl.ds(off, CH)],
                                g_v.at[pl.ds(0, CH)])
                pltpu.sync_copy(g_v.at[pl.ds(0, CH)],
                                s_sh.at[pl.ds(off, CH)])

            @pl.when(cix == 63)
            def _():
                off = 63 * CH
                pltpu.sync_copy(scores_hbm.at[pl.ds(off, VOCAB - 63 * CH)],
                                g_v.at[pl.ds(0, VOCAB - 63 * CH)])
                pltpu.sync_copy(g_v.at[pl.ds(0, VOCAB - 63 * CH)],
                                s_sh.at[pl.ds(off, VOCAB - 63 * CH)])
        pltpu.sync_copy(xT_hbm.at[:, pl.ds(wid * SENT_PER_W, SENT_PER_W)],
                        idx_v)
        plsc.subcore_barrier()

        @pl.loop(0, NCHUNK, step=FLIGHT)
        def _(o):
            copies = []
            for b in range(FLIGHT):
                t = o + b
                w, c = t // CPW, t % CPW
                copies.append(pltpu.async_copy(
                    s_sh.at[idx_v.at[w, pl.ds(c * CHUNK, CHUNK)]],
                    g_v.at[pl.ds(t * CHUNK, CHUNK)],
                    sem,
                ))
            for cp in copies:
                cp.wait()

        cvec = c_v[...]

        @pl.loop(0, GROUPS_PER_W)
        def _(gr):
            acc = cvec
            for w in range(SEQ):
                acc = acc + g_v[pl.ds(gr * L + w * SENT_PER_W, L)]
            out_v[pl.ds(gr * L, L)] = 1.0 / (1.0 + jnp.exp(-acc))

        pltpu.sync_copy(out_v, out_hbm.at[pl.ds(wid * SENT_PER_W, SENT_PER_W)])

    return k(scores, xT, c16)


def kernel(x, table, W_in, b_in, W_fc, b_fc):
    v_col, c16 = _fold_weights(W_in, b_in, W_fc, b_fc)
    scores = _matvec(table.T, v_col)
    return _gather_reduce(scores, x.T, c16)


# speedup vs baseline: 1.2161x; 1.0004x over previous
import functools

import jax
import jax.numpy as jnp
from jax import lax
from jax.experimental import pallas as pl
from jax.experimental.pallas import tpu as pltpu
from jax.experimental.pallas import tpu_sc as plsc

VOCAB = 1_000_000
EMBED = 300
HIDDEN = 256
SEQ = 60
BATCH = 16384

NC = 2
NS = 16
L = 16
NW = NC * NS
SENT_PER_W = BATCH // NW
GROUPS_PER_W = SENT_PER_W // L
IDX_PER_W = SENT_PER_W * SEQ
CHUNK = 128
CPW = SENT_PER_W // CHUNK
NCHUNK = IDX_PER_W // CHUNK
FLIGHT = 16


def _matvec(tableT, W_in, b_in, W_fc, b_fc):
    C = 16384
    G = -(-VOCAB // C)

    def body(t_ref, wi_ref, bi_ref, wf_ref, bf_ref, s_ref, v_s, c_s):
        @pl.when(pl.program_id(0) == 0)
        def _():
            wf = wf_ref[...]
            v_s[...] = jnp.dot(wi_ref[...], wf,
                               preferred_element_type=jnp.float32) * (1.0 / SEQ)
            c_s[0] = (jnp.sum(bi_ref[...] * wf[:, 0]) + bf_ref[0]) * (1.0 / SEQ)

        s_ref[...] = jnp.sum(t_ref[...] * v_s[...], axis=0) + c_s[0]

    return pl.pallas_call(
        body,
        grid=(G,),
        in_specs=[pl.BlockSpec((EMBED, C), lambda i: (0, i)),
                  pl.BlockSpec((EMBED, HIDDEN), lambda i: (0, 0)),
                  pl.BlockSpec((HIDDEN,), lambda i: (0,)),
                  pl.BlockSpec((HIDDEN, 1), lambda i: (0, 0)),
                  pl.BlockSpec((1,), lambda i: (0,))],
        out_specs=pl.BlockSpec((C,), lambda i: (i,)),
        out_shape=jax.ShapeDtypeStruct((VOCAB,), jnp.float32),
        scratch_shapes=[pltpu.VMEM((EMBED, 1), jnp.float32),
                        pltpu.SMEM((1,), jnp.float32)],
        compiler_params=pltpu.CompilerParams(
            dimension_semantics=("arbitrary",)),
    )(tableT, W_in, b_in, W_fc, b_fc)


def _gather_reduce(scores, xT):
    mesh = plsc.VectorSubcoreMesh(core_axis_name="c", subcore_axis_name="s")

    @functools.partial(
        pl.kernel,
        out_type=jax.ShapeDtypeStruct((BATCH,), jnp.float32),
        mesh=mesh,
        scratch_types=[
            pltpu.VMEM((SEQ, SENT_PER_W), jnp.int32),
            pltpu.VMEM((IDX_PER_W,), jnp.float32),
            pltpu.VMEM((SENT_PER_W,), jnp.float32),
            pltpu.VMEM_SHARED((VOCAB,), jnp.float32),
            pltpu.SemaphoreType.DMA,
        ],
    )
    def k(scores_hbm, xT_hbm, out_hbm, idx_v, g_v, out_v, s_sh, sem):
        sid = lax.axis_index("s")
        wid = sid * NC + lax.axis_index("c")

        CH = 15624
        for kk in range(4):
            cix = sid + NS * kk

            @pl.when(cix < 63)
            def _():
                off = cix * CH
                pltpu.sync_copy(scores_hbm.at[pl.ds(off, CH)],
                                g_v.at[pl.ds(0, CH)])
                pltpu.sync_copy(g_v.at[pl.ds(0, CH)],
                                s_sh.at[pl.ds(off, CH)])

            @pl.when(cix == 63)
            def _():
                off = 63 * CH
                pltpu.sync_copy(scores_hbm.at[pl.ds(off, VOCAB - 63 * CH)],
                                g_v.at[pl.ds(0, VOCAB - 63 * CH)])
                pltpu.sync_copy(g_v.at[pl.ds(0, VOCAB - 63 * CH)],
                                s_sh.at[pl.ds(off, VOCAB - 63 * CH)])
        pltpu.sync_copy(xT_hbm.at[:, pl.ds(wid * SENT_PER_W, SENT_PER_W)],
                        idx_v)
        plsc.subcore_barrier()

        @pl.loop(0, NCHUNK, step=FLIGHT)
        def _(o):
            copies = []
            for b in range(FLIGHT):
                t = o + b
                w, c = t // CPW, t % CPW
                copies.append(pltpu.async_copy(
                    s_sh.at[idx_v.at[w, pl.ds(c * CHUNK, CHUNK)]],
                    g_v.at[pl.ds(t * CHUNK, CHUNK)],
                    sem,
                ))
            for cp in copies:
                cp.wait()

        @pl.loop(0, GROUPS_PER_W)
        def _(gr):
            acc = g_v[pl.ds(gr * L, L)]
            for w in range(1, SEQ):
                acc = acc + g_v[pl.ds(gr * L + w * SENT_PER_W, L)]
            out_v[pl.ds(gr * L, L)] = 1.0 / (1.0 + jnp.exp(-acc))

        pltpu.sync_copy(out_v, out_hbm.at[pl.ds(wid * SENT_PER_W, SENT_PER_W)])

    return k(scores, xT)


def kernel(x, table, W_in, b_in, W_fc, b_fc):
    scores = _matvec(table.T, W_in, b_in, W_fc, b_fc)
    return _gather_reduce(scores, x.T)


# speedup vs baseline: 1.2197x; 1.0030x over previous
import functools

import jax
import jax.numpy as jnp
from jax import lax
from jax.experimental import pallas as pl
from jax.experimental.pallas import tpu as pltpu
from jax.experimental.pallas import tpu_sc as plsc

VOCAB = 1_000_000
VOCAB_PAD = 1_000_448
EMBED = 300
HIDDEN = 256
SEQ = 60
BATCH = 16384

NC = 2
NS = 16
L = 16
NW = NC * NS
SENT_PER_W = BATCH // NW
GROUPS_PER_W = SENT_PER_W // L
IDX_PER_W = SENT_PER_W * SEQ
CHUNK = 128
CPW = SENT_PER_W // CHUNK
NCHUNK = IDX_PER_W // CHUNK
FLIGHT = 16


def _matvec(tableT, W_in, b_in, W_fc, b_fc):
    C = 16384
    G = -(-VOCAB_PAD // C)

    def body(t_ref, wi_ref, bi_ref, wf_ref, bf_ref, s_ref, v_s, c_s):
        @pl.when(pl.program_id(0) == 0)
        def _():
            wf = wf_ref[...]
            v_s[...] = jnp.dot(wi_ref[...], wf,
                               preferred_element_type=jnp.float32) * (1.0 / SEQ)
            c_s[0] = (jnp.sum(bi_ref[...] * wf[:, 0]) + bf_ref[0]) * (1.0 / SEQ)

        s_ref[...] = jnp.sum(t_ref[...] * v_s[...], axis=0) + c_s[0]

    return pl.pallas_call(
        body,
        grid=(G,),
        in_specs=[pl.BlockSpec((EMBED, C), lambda i: (0, i)),
                  pl.BlockSpec((EMBED, HIDDEN), lambda i: (0, 0)),
                  pl.BlockSpec((HIDDEN,), lambda i: (0,)),
                  pl.BlockSpec((HIDDEN, 1), lambda i: (0, 0)),
                  pl.BlockSpec((1,), lambda i: (0,))],
        out_specs=pl.BlockSpec((C,), lambda i: (i,)),
        out_shape=jax.ShapeDtypeStruct((VOCAB_PAD,), jnp.float32),
        scratch_shapes=[pltpu.VMEM((EMBED, 1), jnp.float32),
                        pltpu.SMEM((1,), jnp.float32)],
        compiler_params=pltpu.CompilerParams(
            dimension_semantics=("arbitrary",)),
    )(tableT, W_in, b_in, W_fc, b_fc)


def _gather_reduce(scores, xT):
    mesh = plsc.VectorSubcoreMesh(core_axis_name="c", subcore_axis_name="s")

    @functools.partial(
        pl.kernel,
        out_type=jax.ShapeDtypeStruct((BATCH,), jnp.float32),
        mesh=mesh,
        scratch_types=[
            pltpu.VMEM((SEQ, SENT_PER_W), jnp.int32),
            pltpu.VMEM((IDX_PER_W,), jnp.float32),
            pltpu.VMEM((SENT_PER_W,), jnp.float32),
            pltpu.VMEM_SHARED((VOCAB_PAD,), jnp.float32),
            pltpu.SemaphoreType.DMA,
            pltpu.SemaphoreType.DMA,
        ],
    )
    def k(scores_hbm, xT_hbm, out_hbm, idx_v, g_v, out_v, s_sh, sem, sem2):
        sid = lax.axis_index("s")
        wid = sid * NC + lax.axis_index("c")

        idx_cp = pltpu.async_copy(
            xT_hbm.at[:, pl.ds(wid * SENT_PER_W, SENT_PER_W)], idx_v, sem2)

        CH = VOCAB_PAD // 128
        NSTG = 8

        def off(kk):
            return (sid * NSTG + kk) * CH

        bufs = [0, CH]
        h2v = pltpu.async_copy(scores_hbm.at[pl.ds(off(0), CH)],
                               g_v.at[pl.ds(bufs[0], CH)], sem)
        for kk in range(NSTG):
            h2v.wait()
            if kk + 1 < NSTG:
                h2v = pltpu.async_copy(
                    scores_hbm.at[pl.ds(off(kk + 1), CH)],
                    g_v.at[pl.ds(bufs[(kk + 1) % 2], CH)], sem)
            pltpu.sync_copy(g_v.at[pl.ds(bufs[kk % 2], CH)],
                            s_sh.at[pl.ds(off(kk), CH)])
        idx_cp.wait()
        plsc.subcore_barrier()

        @pl.loop(0, NCHUNK, step=FLIGHT)
        def _(o):
            copies = []
            for b in range(FLIGHT):
                t = o + b
                w, c = t // CPW, t % CPW
                copies.append(pltpu.async_copy(
                    s_sh.at[idx_v.at[w, pl.ds(c * CHUNK, CHUNK)]],
                    g_v.at[pl.ds(t * CHUNK, CHUNK)],
                    sem,
                ))
            for cp in copies:
                cp.wait()

        @pl.loop(0, GROUPS_PER_W)
        def _(gr):
            acc = g_v[pl.ds(gr * L, L)]
            for w in range(1, SEQ):
                acc = acc + g_v[pl.ds(gr * L + w * SENT_PER_W, L)]
            out_v[pl.ds(gr * L, L)] = 1.0 / (1.0 + jnp.exp(-acc))

        pltpu.sync_copy(out_v, out_hbm.at[pl.ds(wid * SENT_PER_W, SENT_PER_W)])

    return k(scores, xT)


def kernel(x, table, W_in, b_in, W_fc, b_fc):
    scores = _matvec(table.T, W_in, b_in, W_fc, b_fc)
    return _gather_reduce(scores, x.T)
